# Initial kernel scaffold; baseline (speedup 1.0000x reference)
import functools

import jax
import jax.numpy as jnp
from jax import lax
from jax.experimental import pallas as pl
from jax.experimental.pallas import tpu as pltpu
from jax.experimental.pallas import tpu_sc as plsc

_ALPHA = 0.1
_K = 10
_L = 16
_NC = 2
_NS = 16
_NW = _NC * _NS
_CHUNK = 128


@functools.lru_cache(maxsize=None)
def _make_step(N, D, nch):
    mesh = plsc.VectorSubcoreMesh(core_axis_name="c", subcore_axis_name="s")
    rows_per_tile = N // _NS

    @functools.partial(
        pl.kernel,
        out_type=jax.ShapeDtypeStruct((_NC, N, D), jnp.float32),
        mesh=mesh,
        scratch_types=[
            pltpu.VMEM((_CHUNK,), jnp.int32),
            pltpu.VMEM((1, _CHUNK), jnp.int32),
            pltpu.VMEM((_CHUNK,), jnp.float32),
            pltpu.VMEM((_CHUNK, D), jnp.float32),
            pltpu.VMEM_SHARED((N, D), jnp.float32),
            pltpu.SemaphoreType.DMA,
        ],
    )
    def step(x_hbm, src_hbm, dst_hbm, w_hbm, init_hbm, out_hbm,
             src_v, dst_v, w_v, rows_v, agg_sh, sem):
        c = lax.axis_index("c")
        s = lax.axis_index("s")
        wid = c * _NS + s
        r0 = s * rows_per_tile

        pltpu.sync_copy(init_hbm.at[pl.ds(r0, rows_per_tile)],
                        agg_sh.at[pl.ds(r0, rows_per_tile)])
        plsc.subcore_barrier()

        @pl.loop(0, nch)
        def _chunk(j):
            pltpu.sync_copy(src_hbm.at[wid, j], src_v)
            pltpu.sync_copy(dst_hbm.at[wid, j], dst_v.at[0])
            pltpu.sync_copy(w_hbm.at[wid, j], w_v)
            pltpu.async_copy(x_hbm.at[src_v], rows_v, sem).wait()

            @pl.loop(0, _CHUNK)
            def _edge(e):
                wb = plsc.load_gather(w_v, [jnp.broadcast_to(e, (_L,))])
                for q in range(D // _L):
                    sl = pl.ds(q * _L, _L)
                    rows_v[e, sl] = rows_v[e, sl] * wb

            pltpu.sync_copy(rows_v, agg_sh.at[dst_v.at[0]], add=True)

        plsc.subcore_barrier()
        pltpu.sync_copy(agg_sh.at[pl.ds(r0, rows_per_tile)],
                        out_hbm.at[c, pl.ds(r0, rows_per_tile)])

    return step


def kernel(x, adj_weight, adj_index):
    N, D = x.shape
    E = adj_weight.shape[0]
    src = adj_index[0]
    dst = adj_index[1]

    nch = -(-E // (_NW * _CHUNK))
    pad = _NW * nch * _CHUNK - E
    srcp = jnp.concatenate(
        [src, jnp.zeros((pad,), jnp.int32)]).reshape(_NW, nch, _CHUNK)
    dstp = jnp.concatenate(
        [dst, jnp.zeros((pad,), jnp.int32)]).reshape(_NW, nch, _CHUNK)
    wp = jnp.concatenate(
        [(1.0 - _ALPHA) * adj_weight,
         jnp.zeros((pad,), jnp.float32)]).reshape(_NW, nch, _CHUNK)
    init = (0.5 * _ALPHA) * x

    step = _make_step(N, D, nch)
    xc = x
    for _ in range(_K):
        p = step(xc, srcp, dstp, wp, init)
        xc = p[0] + p[1]
    return xc


# speedup vs baseline: 2.9692x; 2.9692x over previous
import dataclasses
import functools

import jax
import jax.numpy as jnp
from jax import lax
from jax.experimental import pallas as pl
from jax.experimental.pallas import tpu as pltpu
from jax.experimental.pallas import tpu_sc as plsc

_ALPHA = 0.1
_K = 10
_L = 16
_NC = 2
_NS = 16
_NW = _NC * _NS
_CHUNK = 128


@functools.lru_cache(maxsize=None)
def _make_step(Npad, D, nch):
    mesh = plsc.VectorSubcoreMesh(core_axis_name="c", subcore_axis_name="s")
    rows_per_tile = Npad // _NS

    cp = pltpu.CompilerParams()
    if "needs_layout_passes" in pltpu.CompilerParams.__dataclass_fields__:
        cp = dataclasses.replace(cp, needs_layout_passes=False)

    @functools.partial(
        pl.kernel,
        out_type=jax.ShapeDtypeStruct((_NC, Npad, D), jnp.float32),
        mesh=mesh,
        compiler_params=cp,
        scratch_types=[
            pltpu.VMEM((_CHUNK,), jnp.int32),
            pltpu.VMEM((1, _CHUNK), jnp.int32),
            pltpu.VMEM((_CHUNK,), jnp.float32),
            pltpu.VMEM((_CHUNK, D), jnp.float32),
            pltpu.VMEM_SHARED((Npad, D), jnp.float32),
            pltpu.SemaphoreType.DMA,
        ],
    )
    def step(x_hbm, src_hbm, dst_hbm, w_hbm, init_hbm, out_hbm,
             src_v, dst_v, w_v, rows_v, agg_sh, sem):
        c = lax.axis_index("c")
        s = lax.axis_index("s")
        wid = c * _NS + s
        r0 = s * rows_per_tile

        pltpu.sync_copy(init_hbm.at[pl.ds(r0, rows_per_tile)],
                        agg_sh.at[pl.ds(r0, rows_per_tile)])
        plsc.subcore_barrier()

        @pl.loop(0, nch)
        def _chunk(j):
            pltpu.sync_copy(src_hbm.at[wid, j], src_v)
            pltpu.sync_copy(dst_hbm.at[wid, j], dst_v.at[0])
            pltpu.sync_copy(w_hbm.at[wid, j], w_v)
            pltpu.async_copy(x_hbm.at[src_v], rows_v, sem).wait()

            @pl.loop(0, _CHUNK)
            def _edge(e):
                wb = plsc.load_gather(w_v, [jnp.broadcast_to(e, (_L,))])
                for q in range(D // _L):
                    sl = pl.ds(q * _L, _L)
                    rows_v[e, sl] = rows_v[e, sl] * wb

            pltpu.sync_copy(rows_v, agg_sh.at[dst_v.at[0]], add=True)

        plsc.subcore_barrier()
        pltpu.sync_copy(agg_sh.at[pl.ds(r0, rows_per_tile)],
                        out_hbm.at[c, pl.ds(r0, rows_per_tile)])

    return step


def kernel(x, adj_weight, adj_index):
    N, D = x.shape
    E = adj_weight.shape[0]
    src = adj_index[0]
    dst = adj_index[1]

    nch = -(-E // (_NW * _CHUNK))
    pad = _NW * nch * _CHUNK - E
    srcp = jnp.concatenate(
        [src, jnp.zeros((pad,), jnp.int32)]).reshape(_NW, nch, _CHUNK)
    dstp = jnp.concatenate(
        [dst, jnp.zeros((pad,), jnp.int32)]).reshape(_NW, nch, _CHUNK)
    wp = jnp.concatenate(
        [(1.0 - _ALPHA) * adj_weight,
         jnp.zeros((pad,), jnp.float32)]).reshape(_NW, nch, _CHUNK)
    npad_to = _NS * 8
    Npad = -(-N // npad_to) * npad_to
    xc = jnp.pad(x, ((0, Npad - N), (0, 0)))
    init = (0.5 * _ALPHA) * xc

    step = _make_step(Npad, D, nch)
    for _ in range(_K):
        p = step(xc, srcp, dstp, wp, init)
        xc = p[0] + p[1]
    return xc[:N]


# speedup vs baseline: 5.2233x; 1.7592x over previous
import dataclasses
import functools

import jax
import jax.numpy as jnp
from jax import lax
from jax.experimental import pallas as pl
from jax.experimental.pallas import tpu as pltpu
from jax.experimental.pallas import tpu_sc as plsc

_ALPHA = 0.1
_K = 10
_L = 16
_NC = 2
_NS = 16
_CHUNK = 64


@functools.lru_cache(maxsize=None)
def _make_step(Npad, DC, nch):
    assert nch % 4 == 0
    mesh = plsc.VectorSubcoreMesh(core_axis_name="c", subcore_axis_name="s")
    rows_per_tile = Npad // _NS

    cp = pltpu.CompilerParams()
    for _field, _val in (("needs_layout_passes", False),
                         ("use_tc_tiling_on_sc", False)):
        if _field in pltpu.CompilerParams.__dataclass_fields__:
            cp = dataclasses.replace(cp, **{_field: _val})

    @functools.partial(
        pl.kernel,
        out_type=jax.ShapeDtypeStruct((_NC * Npad, DC), jnp.float32),
        mesh=mesh,
        compiler_params=cp,
        scratch_types=[
            pltpu.VMEM((nch + 2, _CHUNK), jnp.int32),
            pltpu.VMEM((nch + 2, _CHUNK), jnp.int32),
            pltpu.VMEM((nch + 2, _CHUNK), jnp.float32),
            pltpu.VMEM((_CHUNK, DC), jnp.float32),
            pltpu.VMEM((_CHUNK, DC), jnp.float32),
            pltpu.VMEM((_CHUNK, DC), jnp.float32),
            pltpu.VMEM((_CHUNK, DC), jnp.float32),
            pltpu.VMEM_SHARED((Npad, DC), jnp.float32),
            pltpu.SemaphoreType.DMA,
            pltpu.SemaphoreType.DMA,
            pltpu.SemaphoreType.DMA,
            pltpu.SemaphoreType.DMA,
        ],
    )
    def step(x_hbm, src_hbm, dst_hbm, w_hbm, init_hbm, out_hbm,
             src_a, dst_a, w_a, rb0, rb1, rb2, rb3, agg_sh, g0, g1, s0, s1):
        c = lax.axis_index("c")
        s = lax.axis_index("s")
        row0 = s * rows_per_tile
        out0 = c * Npad + row0
        bufs = (rb0, rb1, rb2, rb3)
        gsem = (g0, g1)
        ssem = (s0, s1)

        pltpu.sync_copy(init_hbm.at[pl.ds(out0, rows_per_tile)],
                        agg_sh.at[pl.ds(row0, rows_per_tile)])
        pltpu.sync_copy(src_hbm.at[c, s], src_a)
        pltpu.sync_copy(dst_hbm.at[s], dst_a)
        pltpu.sync_copy(w_hbm.at[s], w_a)
        plsc.subcore_barrier()

        def gather(t, buf, sem):
            return pltpu.make_async_copy(x_hbm.at[src_a.at[t]], buf, sem)

        def scatter(t, buf, sem):
            return pltpu.make_async_copy(buf, agg_sh.at[dst_a.at[t]], sem)

        gather(0, rb0, g0).start()
        gather(1, rb1, g1).start()

        @pl.loop(0, nch, step=4)
        def _slot4(j):
            for b in range(4):
                t = j + b
                buf = bufs[b]
                buf2 = bufs[(b + 2) % 4]
                gs = gsem[b % 2]
                ss = ssem[b % 2]

                gather(t, buf, gs).wait()

                @pl.loop(0, _CHUNK, step=4)
                def _scale(e0):
                    for u in range(4):
                        e = e0 + u
                        wb = plsc.load_gather(
                            w_a, [jnp.broadcast_to(t, (_L,)),
                                  jnp.broadcast_to(e, (_L,))])
                        for q in range(DC // _L):
                            sl = pl.ds(q * _L, _L)
                            buf[e, sl] = buf[e, sl] * wb

                if b < 2:
                    @pl.when(j >= 2)
                    def _drain():
                        scatter(t - 2, buf2, ss).wait()
                else:
                    scatter(t - 2, buf2, ss).wait()
                gather(t + 2, buf2, gs).start()
                scatter(t, buf, ss).start(add=True)

        gather(nch, rb0, g0).wait()
        gather(nch + 1, rb1, g1).wait()
        scatter(nch - 2, rb2, s0).wait()
        scatter(nch - 1, rb3, s1).wait()

        plsc.subcore_barrier()
        pltpu.sync_copy(agg_sh.at[pl.ds(row0, rows_per_tile)],
                        out_hbm.at[pl.ds(out0, rows_per_tile)])

    return step


def kernel(x, adj_weight, adj_index):
    N, D = x.shape
    E = adj_weight.shape[0]
    DC = D // _NC
    src = adj_index[0]
    dst = adj_index[1]

    npad_to = _NS * 8
    Npad = -(-N // npad_to) * npad_to

    nch = -(-E // (_NS * _CHUNK))
    nch = -(-nch // 4) * 4
    pad = _NS * nch * _CHUNK - E

    def lay_out(v, fill_dtype):
        body = jnp.concatenate(
            [v, jnp.zeros((pad,), fill_dtype)]).reshape(_NS, nch, _CHUNK)
        tail = jnp.zeros((_NS, 2, _CHUNK), fill_dtype)
        return jnp.concatenate([body, tail], axis=1)

    srcp0 = lay_out(src, jnp.int32)
    srcp = jnp.stack([srcp0, srcp0 + Npad])
    dstp = lay_out(dst, jnp.int32)
    wp = lay_out((1.0 - _ALPHA) * adj_weight, jnp.float32)

    xpad = jnp.pad(x, ((0, Npad - N), (0, 0)))
    xs = jnp.concatenate([xpad[:, :DC], xpad[:, DC:]], axis=0)
    init = _ALPHA * xs

    step = _make_step(Npad, DC, nch)
    for _ in range(_K):
        xs = step(xs, srcp, dstp, wp, init)
    return jnp.concatenate([xs[:N], xs[Npad:Npad + N]], axis=1)


# speedup vs baseline: 7.6181x; 1.4585x over previous
import dataclasses
import functools

import jax
import jax.numpy as jnp
from jax import lax
from jax.experimental import pallas as pl
from jax.experimental.pallas import tpu as pltpu
from jax.experimental.pallas import tpu_sc as plsc

_ALPHA = 0.1
_K = 10
_L = 16
_NC = 2
_NS = 16
_CHUNK = 128


@functools.lru_cache(maxsize=None)
def _make_step(Npad, DC, nch):
    assert nch % 4 == 0
    mesh = plsc.VectorSubcoreMesh(core_axis_name="c", subcore_axis_name="s")
    rows_per_tile = Npad // _NS
    nslot = nch + 2

    cp = pltpu.CompilerParams()
    for _field, _val in (("needs_layout_passes", False),
                         ("use_tc_tiling_on_sc", False)):
        if _field in pltpu.CompilerParams.__dataclass_fields__:
            cp = dataclasses.replace(cp, **{_field: _val})

    @functools.partial(
        pl.kernel,
        out_type=jax.ShapeDtypeStruct((_NC * Npad, DC), jnp.float32),
        mesh=mesh,
        compiler_params=cp,
        scratch_types=(
            [pltpu.VMEM((3, _CHUNK), jnp.int32) for _ in range(4)]
            + [pltpu.VMEM((_CHUNK, DC), jnp.float32) for _ in range(4)]
            + [pltpu.VMEM_SHARED((Npad, DC), jnp.float32),
               pltpu.VMEM_SHARED((Npad, DC), jnp.float32)]
            + [pltpu.SemaphoreType.DMA] * 10
        ),
    )
    def step(x_hbm, ed_hbm, init_hbm, out_hbm,
             ib0, ib1, ib2, ib3, st0, st1, st2, st3, xsh, agg_sh,
             i0, i1, i2, i3, g0, g1, g2, g3, s0, s1):
        c = lax.axis_index("c")
        s = lax.axis_index("s")
        row0 = s * rows_per_tile
        out0 = c * Npad + row0
        cbase = s * nslot
        ibufs = (ib0, ib1, ib2, ib3)
        stage = (st0, st1, st2, st3)
        isem = (i0, i1, i2, i3)
        gsem = (g0, g1, g2, g3)
        ssem = (s0, s1)

        pltpu.sync_copy(x_hbm.at[pl.ds(out0, rows_per_tile)],
                        xsh.at[pl.ds(row0, rows_per_tile)])
        pltpu.sync_copy(init_hbm.at[pl.ds(out0, rows_per_tile)],
                        agg_sh.at[pl.ds(row0, rows_per_tile)])
        plsc.subcore_barrier()

        def load_chunk(t, b):
            return pltpu.make_async_copy(ed_hbm.at[cbase + t], ibufs[b],
                                         isem[b])

        def gather(b):
            return pltpu.make_async_copy(xsh.at[ibufs[b].at[0]], stage[b],
                                         gsem[b])

        def scatter(b, ss):
            return pltpu.make_async_copy(stage[b], agg_sh.at[ibufs[b].at[1]],
                                         ss)

        load_chunk(0, 0).start()
        load_chunk(1, 1).start()
        load_chunk(0, 0).wait()
        gather(0).start()

        @pl.loop(0, nch, step=4)
        def _slot4(j):
            for b in range(4):
                t = j + b
                b1 = (b + 1) % 4
                b2 = (b + 2) % 4
                ss = ssem[b % 2]

                gather(b).wait()

                @pl.loop(0, _CHUNK, step=_L)
                def _scale(e0):
                    wv = plsc.bitcast(ibufs[b][2, pl.ds(e0, _L)], jnp.float32)
                    for u in range(_L):
                        wb = jnp.broadcast_to(wv[u], (_L,))
                        e = e0 + u
                        for q in range(DC // _L):
                            sl = pl.ds(q * _L, _L)
                            stage[b][e, sl] = stage[b][e, sl] * wb

                if b < 2:
                    @pl.when(j >= 2)
                    def _drain():
                        scatter(b2, ss).wait()
                else:
                    scatter(b2, ss).wait()
                load_chunk(t + 2, b2).start()
                load_chunk(t + 1, b1).wait()
                gather(b1).start()
                scatter(b, ss).start(add=True)

        load_chunk(nch + 1, (nch + 1) % 4).wait()
        gather(nch % 4).wait()
        scatter((nch - 2) % 4, ssem[nch % 2]).wait()
        scatter((nch - 1) % 4, ssem[(nch + 1) % 2]).wait()

        plsc.subcore_barrier()
        pltpu.sync_copy(agg_sh.at[pl.ds(row0, rows_per_tile)],
                        out_hbm.at[pl.ds(out0, rows_per_tile)])

    return step


def kernel(x, adj_weight, adj_index):
    N, D = x.shape
    E = adj_weight.shape[0]
    DC = D // _NC
    src = adj_index[0]
    dst = adj_index[1]

    npad_to = _NS * 8
    Npad = -(-N // npad_to) * npad_to

    nch = -(-E // (_NS * _CHUNK))
    nch = -(-nch // 4) * 4
    pad = _NS * nch * _CHUNK - E

    def lay_out(v, fill_dtype):
        return jnp.concatenate(
            [v, jnp.zeros((pad,), fill_dtype)]).reshape(_NS, nch, _CHUNK)

    w_i = jax.lax.bitcast_convert_type(
        ((1.0 - _ALPHA) * adj_weight).astype(jnp.float32), jnp.int32)
    packed = jnp.stack(
        [lay_out(src, jnp.int32), lay_out(dst, jnp.int32),
         lay_out(w_i, jnp.int32)], axis=2)
    tail = jnp.zeros((_NS, 2, 3, _CHUNK), jnp.int32)
    ed = jnp.concatenate([packed, tail], axis=1).reshape(
        _NS * (nch + 2), 3, _CHUNK)

    xpad = jnp.pad(x, ((0, Npad - N), (0, 0)))
    xs = jnp.concatenate([xpad[:, :DC], xpad[:, DC:]], axis=0)
    init = _ALPHA * xs

    step = _make_step(Npad, DC, nch)
    for _ in range(_K):
        xs = step(xs, ed, init)
    return jnp.concatenate([xs[:N], xs[Npad:Npad + N]], axis=1)


# speedup vs baseline: 7.9617x; 1.0451x over previous
import dataclasses
import functools

import jax
import jax.numpy as jnp
from jax import lax
from jax.experimental import pallas as pl
from jax.experimental.pallas import tpu as pltpu
from jax.experimental.pallas import tpu_sc as plsc

_ALPHA = 0.1
_K = 10
_L = 16
_NC = 2
_NS = 16
_CHUNK = 128


@functools.lru_cache(maxsize=None)
def _make_step(Npad, DC, nch):
    assert nch % 4 == 0
    mesh = plsc.VectorSubcoreMesh(core_axis_name="c", subcore_axis_name="s")
    rows_per_tile = Npad // _NS
    nslot = nch + 2

    cp = pltpu.CompilerParams()
    for _field, _val in (("needs_layout_passes", False),
                         ("use_tc_tiling_on_sc", False)):
        if _field in pltpu.CompilerParams.__dataclass_fields__:
            cp = dataclasses.replace(cp, **{_field: _val})

    @functools.partial(
        pl.kernel,
        out_type=jax.ShapeDtypeStruct((_NC * Npad, DC), jnp.float32),
        mesh=mesh,
        compiler_params=cp,
        scratch_types=(
            [pltpu.VMEM((3, _CHUNK), jnp.int32) for _ in range(4)]
            + [pltpu.VMEM((_CHUNK, DC), jnp.float32) for _ in range(4)]
            + [pltpu.VMEM_SHARED((Npad, DC), jnp.float32),
               pltpu.VMEM_SHARED((Npad, DC), jnp.float32)]
            + [pltpu.SemaphoreType.DMA] * 10
        ),
    )
    def step(x_hbm, ed_hbm, init_hbm, out_hbm,
             ib0, ib1, ib2, ib3, st0, st1, st2, st3, xsh, agg_sh,
             i0, i1, i2, i3, g0, g1, g2, g3, s0, s1):
        c = lax.axis_index("c")
        s = lax.axis_index("s")
        row0 = s * rows_per_tile
        out0 = c * Npad + row0
        cbase = s * nslot
        ibufs = (ib0, ib1, ib2, ib3)
        stage = (st0, st1, st2, st3)
        isem = (i0, i1, i2, i3)
        gsem = (g0, g1, g2, g3)
        ssem = (s0, s1)

        pltpu.sync_copy(x_hbm.at[pl.ds(out0, rows_per_tile)],
                        xsh.at[pl.ds(row0, rows_per_tile)])
        pltpu.sync_copy(init_hbm.at[pl.ds(out0, rows_per_tile)],
                        agg_sh.at[pl.ds(row0, rows_per_tile)])
        plsc.subcore_barrier()

        def load_chunk(t, b):
            return pltpu.make_async_copy(ed_hbm.at[cbase + t], ibufs[b],
                                         isem[b])

        def gather(b):
            return pltpu.make_async_copy(xsh.at[ibufs[b].at[0]], stage[b],
                                         gsem[b])

        def scatter(b, ss):
            return pltpu.make_async_copy(stage[b], agg_sh.at[ibufs[b].at[1]],
                                         ss)

        load_chunk(0, 0).start()
        load_chunk(1, 1).start()
        load_chunk(0, 0).wait()
        gather(0).start()

        @pl.loop(0, nch, step=4)
        def _slot4(j):
            for b in range(4):
                t = j + b
                b1 = (b + 1) % 4
                b2 = (b + 2) % 4
                ss = ssem[b % 2]

                gather(b).wait()
                load_chunk(t + 1, b1).wait()
                gather(b1).start()

                @pl.loop(0, _CHUNK, step=_L)
                def _scale(e0):
                    wv = plsc.bitcast(ibufs[b][2, pl.ds(e0, _L)], jnp.float32)
                    for u in range(_L):
                        wb = jnp.broadcast_to(wv[u], (_L,))
                        e = e0 + u
                        for q in range(DC // _L):
                            sl = pl.ds(q * _L, _L)
                            stage[b][e, sl] = stage[b][e, sl] * wb

                if b < 2:
                    @pl.when(j >= 2)
                    def _drain():
                        scatter(b2, ss).wait()
                else:
                    scatter(b2, ss).wait()
                load_chunk(t + 2, b2).start()
                scatter(b, ss).start(add=True)

        load_chunk(nch + 1, (nch + 1) % 4).wait()
        gather(nch % 4).wait()
        scatter((nch - 2) % 4, ssem[nch % 2]).wait()
        scatter((nch - 1) % 4, ssem[(nch + 1) % 2]).wait()

        plsc.subcore_barrier()
        pltpu.sync_copy(agg_sh.at[pl.ds(row0, rows_per_tile)],
                        out_hbm.at[pl.ds(out0, rows_per_tile)])

    return step


def kernel(x, adj_weight, adj_index):
    N, D = x.shape
    E = adj_weight.shape[0]
    DC = D // _NC
    src = adj_index[0]
    dst = adj_index[1]

    npad_to = _NS * 8
    Npad = -(-N // npad_to) * npad_to

    nch = -(-E // (_NS * _CHUNK))
    nch = -(-nch // 4) * 4
    pad = _NS * nch * _CHUNK - E

    def lay_out(v, fill_dtype):
        return jnp.concatenate(
            [v, jnp.zeros((pad,), fill_dtype)]).reshape(_NS, nch, _CHUNK)

    w_i = jax.lax.bitcast_convert_type(
        ((1.0 - _ALPHA) * adj_weight).astype(jnp.float32), jnp.int32)
    packed = jnp.stack(
        [lay_out(src, jnp.int32), lay_out(dst, jnp.int32),
         lay_out(w_i, jnp.int32)], axis=2)
    tail = jnp.zeros((_NS, 2, 3, _CHUNK), jnp.int32)
    ed = jnp.concatenate([packed, tail], axis=1).reshape(
        _NS * (nch + 2), 3, _CHUNK)

    xpad = jnp.pad(x, ((0, Npad - N), (0, 0)))
    xs = jnp.concatenate([xpad[:, :DC], xpad[:, DC:]], axis=0)
    init = _ALPHA * xs

    step = _make_step(Npad, DC, nch)
    for _ in range(_K):
        xs = step(xs, ed, init)
    return jnp.concatenate([xs[:N], xs[Npad:Npad + N]], axis=1)
